# Initial kernel scaffold; baseline (speedup 1.0000x reference)
import jax
import jax.numpy as jnp
from jax.experimental import pallas as pl


def kernel(x, edge_index, batch, edge_attr, pos, Wm1, bm1, Wu1, bu1, Wm2, bm2, Wu2, bu2, Wout, bout):
    raise NotImplementedError("write your pallas kernel here")


# speedup vs baseline: 3.0449x; 3.0449x over previous
import functools

import jax
import jax.numpy as jnp
from jax import lax
from jax.experimental import pallas as pl
from jax.experimental.pallas import tpu as pltpu
from jax.experimental.pallas import tpu_sc as plsc

N_NODES = 10000
N_EDGES = 320000
D = 128
D_EDGE = 16
G = 64
LANES = 16

NW = 32
EPW = N_EDGES // NW
CHUNK = 80
NCHUNK = EPW // CHUNK
NSUB = 16
N_PAD = 10240
RPS = N_PAD // NSUB

_f32 = jnp.float32


def _proj_body(x_ref, ws_ref, wd_ref, bm_ref, a_ref, b_ref):
    x = x_ref[...]
    a_ref[...] = jnp.dot(x, ws_ref[...], preferred_element_type=_f32)
    b_ref[...] = jnp.dot(x, wd_ref[...], preferred_element_type=_f32) + bm_ref[...]


def _node_proj(x, ws, wd, bm):
    return pl.pallas_call(
        _proj_body,
        out_shape=[
            jax.ShapeDtypeStruct((N_NODES, D), _f32),
            jax.ShapeDtypeStruct((N_NODES, D), _f32),
        ],
    )(x, ws, wd, bm.reshape(1, D))


_EROWS = N_EDGES // 8
_EBLK = 1000


def _epr_body(ea_ref, w1_ref, w2_ref, e1_ref, e2_ref):
    ea = ea_ref[...]
    e1_ref[...] = jnp.dot(ea, w1_ref[...], preferred_element_type=_f32)
    e2_ref[...] = jnp.dot(ea, w2_ref[...], preferred_element_type=_f32)


def _edge_proj(edge_attr, we1, we2):
    ea2 = edge_attr.reshape(_EROWS, 8 * D_EDGE)

    def blockdiag(we):
        wb = jnp.zeros((8 * D_EDGE, 8 * D), _f32)
        for k in range(8):
            wb = lax.dynamic_update_slice(wb, we, (D_EDGE * k, D * k))
        return wb

    e1, e2 = pl.pallas_call(
        _epr_body,
        grid=(_EROWS // _EBLK,),
        in_specs=[
            pl.BlockSpec((_EBLK, 8 * D_EDGE), lambda i: (i, 0)),
            pl.BlockSpec((8 * D_EDGE, 8 * D), lambda i: (0, 0)),
            pl.BlockSpec((8 * D_EDGE, 8 * D), lambda i: (0, 0)),
        ],
        out_specs=[
            pl.BlockSpec((_EBLK, 8 * D), lambda i: (i, 0)),
            pl.BlockSpec((_EBLK, 8 * D), lambda i: (i, 0)),
        ],
        out_shape=[
            jax.ShapeDtypeStruct((_EROWS, 8 * D), _f32),
            jax.ShapeDtypeStruct((_EROWS, 8 * D), _f32),
        ],
    )(ea2, blockdiag(we1), blockdiag(we2))
    return e1.reshape(N_EDGES, D), e2.reshape(N_EDGES, D)


def _update_body(x_ref, p_ref, wx_ref, wa_ref, bu_ref, h_ref):
    agg = p_ref[0, :N_NODES] + p_ref[1, :N_NODES]
    h = (jnp.dot(x_ref[...], wx_ref[...], preferred_element_type=_f32)
         + jnp.dot(agg, wa_ref[...], preferred_element_type=_f32)
         + bu_ref[...])
    h_ref[...] = jnp.maximum(h, 0.0)


def _update(x, parts, wx, wa, bu):
    return pl.pallas_call(
        _update_body,
        out_shape=jax.ShapeDtypeStruct((N_NODES, D), _f32),
    )(x, parts.reshape(2, N_PAD, D), wx, wa, bu.reshape(1, D))


def _pool_body(h_ref, batch_ref, wout_ref, bout_ref, o_ref):
    onehot = (batch_ref[...] ==
              lax.broadcasted_iota(jnp.int32, (G, N_NODES), 0)).astype(_f32)
    sums = jnp.dot(onehot, h_ref[...], preferred_element_type=_f32)
    counts = jnp.sum(onehot, axis=1)
    pooled = sums / jnp.maximum(counts, 1.0)[:, None]
    o_ref[...] = jnp.dot(pooled, wout_ref[...], preferred_element_type=_f32) + bout_ref[...]


def _pool_head(h, batch_i32, wout, bout):
    return pl.pallas_call(
        _pool_body,
        out_shape=jax.ShapeDtypeStruct((G, wout.shape[1]), _f32),
    )(h, batch_i32.reshape(1, N_NODES), wout, bout.reshape(1, wout.shape[1]))


@functools.lru_cache(maxsize=1)
def _make_edge_pass():
  mesh = plsc.VectorSubcoreMesh(core_axis_name="c", subcore_axis_name="s")

  @functools.partial(
      pl.kernel,
      out_type=jax.ShapeDtypeStruct((2 * N_PAD, D), _f32),
      mesh=mesh,
      scratch_types=[
          pltpu.VMEM((CHUNK,), jnp.int32),
          pltpu.VMEM((CHUNK,), jnp.int32),
          pltpu.VMEM((CHUNK, D), _f32),
          pltpu.VMEM((CHUNK, D), _f32),
          pltpu.VMEM((CHUNK, D), _f32),
          pltpu.VMEM((CHUNK, D), _f32),
          pltpu.VMEM_SHARED((N_PAD, D), _f32),
          pltpu.SemaphoreType.DMA,
          pltpu.SemaphoreType.DMA,
      ],
  )
  def edge_pass(a_hbm, b_hbm, epr_hbm, src_hbm, dst_hbm, zeros_hbm, out_hbm,
                src_v, dst_v, a_v, b_v, e_v, m_v, agg_sh, sem_a, sem_b):
    c = lax.axis_index("c")
    s = lax.axis_index("s")
    wid = s * 2 + c

    pltpu.sync_copy(zeros_hbm, agg_sh.at[pl.ds(s * RPS, RPS)])
    plsc.subcore_barrier()

    base = wid * EPW

    def chunk_body(ci, carry):
        off = base + ci * CHUNK
        pltpu.sync_copy(src_hbm.at[pl.ds(off, CHUNK)], src_v)
        pltpu.sync_copy(dst_hbm.at[pl.ds(off, CHUNK)], dst_v)
        ga = pltpu.async_copy(a_hbm.at[src_v], a_v, sem_a)
        gb = pltpu.async_copy(b_hbm.at[dst_v], b_v, sem_b)
        pltpu.sync_copy(epr_hbm.at[pl.ds(off, CHUNK)], e_v)
        ga.wait()
        gb.wait()

        def row_body(i, rcarry):
            for j in range(D // LANES):
                sl = pl.ds(j * LANES, LANES)
                m = a_v[i, sl] + b_v[i, sl] + e_v[i, sl]
                m_v[i, sl] = jnp.maximum(m, 0.0)
            return rcarry

        lax.fori_loop(0, CHUNK, row_body, 0)
        pltpu.sync_copy(m_v, agg_sh.at[dst_v], add=True)
        return carry

    lax.fori_loop(0, NCHUNK, chunk_body, 0)
    plsc.subcore_barrier()

    row0 = c * N_PAD + s * RPS
    pltpu.sync_copy(agg_sh.at[pl.ds(s * RPS, RPS)],
                    out_hbm.at[pl.ds(row0, RPS)])

  return edge_pass


def _edge_pass(a, b, epr, src, dst, zeros):
    return _make_edge_pass()(a, b, epr, src, dst, zeros)


def kernel(x, edge_index, batch, edge_attr, pos,
           Wm1, bm1, Wu1, bu1, Wm2, bm2, Wu2, bu2, Wout, bout):
    del pos
    src = edge_index[0].astype(jnp.int32)
    dst = edge_index[1].astype(jnp.int32)
    batch_i32 = batch.astype(jnp.int32)
    zeros = jnp.zeros((RPS, D), _f32)

    ws1, wd1, we1 = Wm1[:D], Wm1[D:2 * D], Wm1[2 * D:]
    ws2, wd2, we2 = Wm2[:D], Wm2[D:2 * D], Wm2[2 * D:]
    wx1, wa1 = Wu1[:D], Wu1[D:]
    wx2, wa2 = Wu2[:D], Wu2[D:]

    epr1, epr2 = _edge_proj(edge_attr, we1, we2)

    a1, b1 = _node_proj(x, ws1, wd1, bm1)
    parts1 = _edge_pass(a1, b1, epr1, src, dst, zeros)
    h1 = _update(x, parts1, wx1, wa1, bu1)

    a2, b2 = _node_proj(h1, ws2, wd2, bm2)
    parts2 = _edge_pass(a2, b2, epr2, src, dst, zeros)
    h2 = _update(h1, parts2, wx2, wa2, bu2)

    return _pool_head(h2, batch_i32, Wout, bout)


# speedup vs baseline: 4.4927x; 1.4755x over previous
import functools

import jax
import jax.numpy as jnp
from jax import lax
from jax.experimental import pallas as pl
from jax.experimental.pallas import tpu as pltpu
from jax.experimental.pallas import tpu_sc as plsc

N_NODES = 10000
N_EDGES = 320000
D = 128
D_EDGE = 16
G = 64
LANES = 16

NW = 32
EPW = N_EDGES // NW
CHUNK = 80
NCHUNK = EPW // CHUNK
NSUB = 16
N_PAD = 10240
RPS = N_PAD // NSUB

_f32 = jnp.float32


def _proj_body(x_ref, ws_ref, wd_ref, bm_ref, a_ref, b_ref):
    x = x_ref[...]
    a_ref[...] = jnp.dot(x, ws_ref[...], preferred_element_type=_f32)
    b_ref[...] = jnp.dot(x, wd_ref[...], preferred_element_type=_f32) + bm_ref[...]


def _node_proj(x, ws, wd, bm):
    return pl.pallas_call(
        _proj_body,
        out_shape=[
            jax.ShapeDtypeStruct((N_NODES, D), _f32),
            jax.ShapeDtypeStruct((N_NODES, D), _f32),
        ],
    )(x, ws, wd, bm.reshape(1, D))


_EROWS = N_EDGES // 8
_EBLK = 1000


def _epr_body(ea_ref, w1_ref, w2_ref, e1_ref, e2_ref):
    ea = ea_ref[...]
    e1_ref[...] = jnp.dot(ea, w1_ref[...], preferred_element_type=_f32)
    e2_ref[...] = jnp.dot(ea, w2_ref[...], preferred_element_type=_f32)


def _edge_proj(edge_attr, we1, we2):
    ea2 = edge_attr.reshape(_EROWS, 8 * D_EDGE)

    def blockdiag(we):
        wb = jnp.zeros((8 * D_EDGE, 8 * D), _f32)
        for k in range(8):
            wb = lax.dynamic_update_slice(wb, we, (D_EDGE * k, D * k))
        return wb

    e1, e2 = pl.pallas_call(
        _epr_body,
        grid=(_EROWS // _EBLK,),
        in_specs=[
            pl.BlockSpec((_EBLK, 8 * D_EDGE), lambda i: (i, 0)),
            pl.BlockSpec((8 * D_EDGE, 8 * D), lambda i: (0, 0)),
            pl.BlockSpec((8 * D_EDGE, 8 * D), lambda i: (0, 0)),
        ],
        out_specs=[
            pl.BlockSpec((_EBLK, 8 * D), lambda i: (i, 0)),
            pl.BlockSpec((_EBLK, 8 * D), lambda i: (i, 0)),
        ],
        out_shape=[
            jax.ShapeDtypeStruct((_EROWS, 8 * D), _f32),
            jax.ShapeDtypeStruct((_EROWS, 8 * D), _f32),
        ],
    )(ea2, blockdiag(we1), blockdiag(we2))
    return e1.reshape(N_EDGES, D), e2.reshape(N_EDGES, D)


def _update_body(x_ref, p_ref, wx_ref, wa_ref, bu_ref, h_ref):
    agg = p_ref[0, :N_NODES] + p_ref[1, :N_NODES]
    h = (jnp.dot(x_ref[...], wx_ref[...], preferred_element_type=_f32)
         + jnp.dot(agg, wa_ref[...], preferred_element_type=_f32)
         + bu_ref[...])
    h_ref[...] = jnp.maximum(h, 0.0)


def _update(x, parts, wx, wa, bu):
    return pl.pallas_call(
        _update_body,
        out_shape=jax.ShapeDtypeStruct((N_NODES, D), _f32),
    )(x, parts.reshape(2, N_PAD, D), wx, wa, bu.reshape(1, D))


def _pool_body(h_ref, batch_ref, wout_ref, bout_ref, o_ref):
    onehot = (batch_ref[...] ==
              lax.broadcasted_iota(jnp.int32, (G, N_NODES), 0)).astype(_f32)
    sums = jnp.dot(onehot, h_ref[...], preferred_element_type=_f32)
    counts = jnp.sum(onehot, axis=1)
    pooled = sums / jnp.maximum(counts, 1.0)[:, None]
    o_ref[...] = jnp.dot(pooled, wout_ref[...], preferred_element_type=_f32) + bout_ref[...]


def _pool_head(h, batch_i32, wout, bout):
    return pl.pallas_call(
        _pool_body,
        out_shape=jax.ShapeDtypeStruct((G, wout.shape[1]), _f32),
    )(h, batch_i32.reshape(1, N_NODES), wout, bout.reshape(1, wout.shape[1]))


@functools.lru_cache(maxsize=1)
def _make_edge_pass():
  mesh = plsc.VectorSubcoreMesh(core_axis_name="c", subcore_axis_name="s")

  ring_types = []
  for _ in range(3):
      ring_types += [pltpu.VMEM((CHUNK,), jnp.int32),
                     pltpu.VMEM((CHUNK,), jnp.int32),
                     pltpu.VMEM((CHUNK, D), _f32)]

  @functools.partial(
      pl.kernel,
      out_type=jax.ShapeDtypeStruct((2 * N_PAD, D), _f32),
      mesh=mesh,
      scratch_types=ring_types + [
          pltpu.VMEM_SHARED((N_PAD, D), _f32),
      ] + [pltpu.SemaphoreType.DMA] * 9,
  )
  def edge_pass(a_hbm, b_hbm, epr_hbm, src_hbm, dst_hbm, zeros_hbm, out_hbm,
                s0, d0, e0, s1, d1, e1, s2, d2, e2,
                agg_sh, gse0, gse1, gse2, ga0, ga1, ga2, ss0, ss1, ss2):
    c = lax.axis_index("c")
    s = lax.axis_index("s")
    wid = s * 2 + c
    base = wid * EPW

    rings = ((s0, d0, e0, gse0, ga0, ss0),
             (s1, d1, e1, gse1, ga1, ss1),
             (s2, d2, e2, gse2, ga2, ss2))

    def issue_head(t, r):
        sv, dv, e, gse, ga, ss = rings[r]
        off = base + t * CHUNK
        pltpu.async_copy(src_hbm.at[pl.ds(off, CHUNK)], sv, gse)
        pltpu.async_copy(dst_hbm.at[pl.ds(off, CHUNK)], dv, gse)
        pltpu.async_copy(epr_hbm.at[pl.ds(off, CHUNK)], e, gse)

    def drain_head(t, r):
        sv, dv, e, gse, ga, ss = rings[r]
        off = base + t * CHUNK
        pltpu.make_async_copy(src_hbm.at[pl.ds(off, CHUNK)], sv, gse).wait()
        pltpu.make_async_copy(dst_hbm.at[pl.ds(off, CHUNK)], dv, gse).wait()
        pltpu.make_async_copy(epr_hbm.at[pl.ds(off, CHUNK)], e, gse).wait()

    def issue_adds(r):
        sv, dv, e, gse, ga, ss = rings[r]
        pltpu.async_copy(a_hbm.at[sv], e, ga, add=True)
        pltpu.async_copy(b_hbm.at[dv], e, ga, add=True)

    def drain_adds(r):
        sv, dv, e, gse, ga, ss = rings[r]
        pltpu.make_async_copy(a_hbm.at[sv], e, ga).wait()
        pltpu.make_async_copy(b_hbm.at[dv], e, ga).wait()

    def compute(r):
        e = rings[r][2]

        def row(i, carry):
            for j in range(D // LANES):
                sl = pl.ds(j * LANES, LANES)
                e[i, sl] = jnp.maximum(e[i, sl], 0.0)
            return carry

        lax.fori_loop(0, CHUNK, row, 0)

    def scatter(r):
        sv, dv, e, gse, ga, ss = rings[r]
        pltpu.async_copy(e, agg_sh.at[dv], ss, add=True)

    def wait_scatter(r):
        sv, dv, e, gse, ga, ss = rings[r]
        pltpu.make_async_copy(e, agg_sh.at[dv], ss).wait()

    def steady(t, r, nxt_adds=True, nxt2_head=True, wait_sc=True):
        drain_adds(r)
        if nxt_adds:
            drain_head(t + 1, (r + 1) % 3)
            issue_adds((r + 1) % 3)
        if nxt2_head:
            if wait_sc:
                wait_scatter((r + 2) % 3)
            issue_head(t + 2, (r + 2) % 3)
        compute(r)
        scatter(r)

    pltpu.sync_copy(zeros_hbm, agg_sh.at[pl.ds(s * RPS, RPS)])
    plsc.subcore_barrier()

    issue_head(0, 0)
    drain_head(0, 0)
    issue_adds(0)
    issue_head(1, 1)
    steady(0, 0, wait_sc=False)
    steady(1, 1)

    def loop_body(k, carry):
        t0 = 2 + 3 * k
        steady(t0, 2)
        steady(t0 + 1, 0)
        steady(t0 + 2, 1)
        return carry

    lax.fori_loop(0, (NCHUNK - 5) // 3, loop_body, 0)
    steady(NCHUNK - 3, 2)
    steady(NCHUNK - 2, 0, nxt2_head=False)
    steady(NCHUNK - 1, 1, nxt_adds=False, nxt2_head=False)
    wait_scatter(2)
    wait_scatter(0)
    wait_scatter(1)
    plsc.subcore_barrier()

    row0 = c * N_PAD + s * RPS
    pltpu.sync_copy(agg_sh.at[pl.ds(s * RPS, RPS)],
                    out_hbm.at[pl.ds(row0, RPS)])

  return edge_pass


def _edge_pass(a, b, epr, src, dst, zeros):
    return _make_edge_pass()(a, b, epr, src, dst, zeros)


def kernel(x, edge_index, batch, edge_attr, pos,
           Wm1, bm1, Wu1, bu1, Wm2, bm2, Wu2, bu2, Wout, bout):
    del pos
    src = edge_index[0].astype(jnp.int32)
    dst = edge_index[1].astype(jnp.int32)
    batch_i32 = batch.astype(jnp.int32)
    zeros = jnp.zeros((RPS, D), _f32)

    ws1, wd1, we1 = Wm1[:D], Wm1[D:2 * D], Wm1[2 * D:]
    ws2, wd2, we2 = Wm2[:D], Wm2[D:2 * D], Wm2[2 * D:]
    wx1, wa1 = Wu1[:D], Wu1[D:]
    wx2, wa2 = Wu2[:D], Wu2[D:]

    epr1, epr2 = _edge_proj(edge_attr, we1, we2)

    a1, b1 = _node_proj(x, ws1, wd1, bm1)
    parts1 = _edge_pass(a1, b1, epr1, src, dst, zeros)
    h1 = _update(x, parts1, wx1, wa1, bu1)

    a2, b2 = _node_proj(h1, ws2, wd2, bm2)
    parts2 = _edge_pass(a2, b2, epr2, src, dst, zeros)
    h2 = _update(h1, parts2, wx2, wa2, bu2)

    return _pool_head(h2, batch_i32, Wout, bout)


# speedup vs baseline: 4.5578x; 1.0145x over previous
import functools

import jax
import jax.numpy as jnp
from jax import lax
from jax.experimental import pallas as pl
from jax.experimental.pallas import tpu as pltpu
from jax.experimental.pallas import tpu_sc as plsc

N_NODES = 10000
N_EDGES = 320000
D = 128
D_EDGE = 16
G = 64
LANES = 16

NW = 32
EPW = N_EDGES // NW
CHUNK = 80
NCHUNK = EPW // CHUNK
NSUB = 16
N_PAD = 10240
RPS = N_PAD // NSUB

_f32 = jnp.float32


def _proj_body(x_ref, ws_ref, wd_ref, bm_ref, a_ref, b_ref):
    x = x_ref[...]
    a_ref[...] = jnp.dot(x, ws_ref[...], preferred_element_type=_f32)
    b_ref[...] = jnp.dot(x, wd_ref[...], preferred_element_type=_f32) + bm_ref[...]


def _node_proj(x, ws, wd, bm):
    return pl.pallas_call(
        _proj_body,
        out_shape=[
            jax.ShapeDtypeStruct((N_NODES, D), _f32),
            jax.ShapeDtypeStruct((N_NODES, D), _f32),
        ],
    )(x, ws, wd, bm.reshape(1, D))


_EROWS = N_EDGES // 8
_EBLK = 1000


def _epr_body(ea_ref, w_ref, e_ref):
    e_ref[...] = jnp.dot(ea_ref[...], w_ref[...], preferred_element_type=_f32)


def _edge_proj(edge_attr, we):
    ea2 = edge_attr.reshape(_EROWS, 8 * D_EDGE)
    wb = jnp.zeros((8 * D_EDGE, 8 * D), _f32)
    for k in range(8):
        wb = lax.dynamic_update_slice(wb, we, (D_EDGE * k, D * k))

    e = pl.pallas_call(
        _epr_body,
        grid=(_EROWS // _EBLK,),
        in_specs=[
            pl.BlockSpec((_EBLK, 8 * D_EDGE), lambda i: (i, 0)),
            pl.BlockSpec((8 * D_EDGE, 8 * D), lambda i: (0, 0)),
        ],
        out_specs=pl.BlockSpec((_EBLK, 8 * D), lambda i: (i, 0)),
        out_shape=jax.ShapeDtypeStruct((_EROWS, 8 * D), _f32),
    )(ea2, wb)
    return e.reshape(N_EDGES, D)


def _update_proj_body(x_ref, p_ref, wx_ref, wa_ref, bu_ref, ws_ref, wd_ref,
                      bm_ref, h_ref, a_ref, b_ref):
    agg = p_ref[0, :N_NODES] + p_ref[1, :N_NODES]
    h = (jnp.dot(x_ref[...], wx_ref[...], preferred_element_type=_f32)
         + jnp.dot(agg, wa_ref[...], preferred_element_type=_f32)
         + bu_ref[...])
    h = jnp.maximum(h, 0.0)
    h_ref[...] = h
    a_ref[...] = jnp.dot(h, ws_ref[...], preferred_element_type=_f32)
    b_ref[...] = jnp.dot(h, wd_ref[...], preferred_element_type=_f32) + bm_ref[...]


def _update_proj(x, parts, wx, wa, bu, ws, wd, bm):
    return pl.pallas_call(
        _update_proj_body,
        out_shape=[
            jax.ShapeDtypeStruct((N_NODES, D), _f32),
            jax.ShapeDtypeStruct((N_NODES, D), _f32),
            jax.ShapeDtypeStruct((N_NODES, D), _f32),
        ],
    )(x, parts.reshape(2, N_PAD, D), wx, wa, bu.reshape(1, D),
      ws, wd, bm.reshape(1, D))


def _update_pool_body(x_ref, p_ref, wx_ref, wa_ref, bu_ref, batch_ref,
                      wout_ref, bout_ref, o_ref):
    agg = p_ref[0, :N_NODES] + p_ref[1, :N_NODES]
    h = (jnp.dot(x_ref[...], wx_ref[...], preferred_element_type=_f32)
         + jnp.dot(agg, wa_ref[...], preferred_element_type=_f32)
         + bu_ref[...])
    h = jnp.maximum(h, 0.0)
    onehot = (batch_ref[...] ==
              lax.broadcasted_iota(jnp.int32, (G, N_NODES), 0)).astype(_f32)
    sums = jnp.dot(onehot, h, preferred_element_type=_f32)
    counts = jnp.sum(onehot, axis=1)
    pooled = sums / jnp.maximum(counts, 1.0)[:, None]
    o_ref[...] = jnp.dot(pooled, wout_ref[...], preferred_element_type=_f32) + bout_ref[...]


def _update_pool(x, parts, wx, wa, bu, batch_i32, wout, bout):
    return pl.pallas_call(
        _update_pool_body,
        out_shape=jax.ShapeDtypeStruct((G, wout.shape[1]), _f32),
    )(x, parts.reshape(2, N_PAD, D), wx, wa, bu.reshape(1, D),
      batch_i32.reshape(1, N_NODES), wout, bout.reshape(1, wout.shape[1]))


@functools.lru_cache(maxsize=1)
def _make_edge_pass():
  mesh = plsc.VectorSubcoreMesh(core_axis_name="c", subcore_axis_name="s")

  ring_types = []
  for _ in range(3):
      ring_types += [pltpu.VMEM((CHUNK,), jnp.int32),
                     pltpu.VMEM((CHUNK,), jnp.int32),
                     pltpu.VMEM((CHUNK, D), _f32)]

  @functools.partial(
      pl.kernel,
      out_type=jax.ShapeDtypeStruct((2 * N_PAD, D), _f32),
      mesh=mesh,
      scratch_types=ring_types + [
          pltpu.VMEM_SHARED((N_PAD, D), _f32),
      ] + [pltpu.SemaphoreType.DMA] * 9,
  )
  def edge_pass(a_hbm, b_hbm, epr_hbm, src_hbm, dst_hbm, zeros_hbm, out_hbm,
                s0, d0, e0, s1, d1, e1, s2, d2, e2,
                agg_sh, gse0, gse1, gse2, ga0, ga1, ga2, ss0, ss1, ss2):
    c = lax.axis_index("c")
    s = lax.axis_index("s")
    wid = s * 2 + c
    base = wid * EPW

    rings = ((s0, d0, e0, gse0, ga0, ss0),
             (s1, d1, e1, gse1, ga1, ss1),
             (s2, d2, e2, gse2, ga2, ss2))

    def issue_head(t, r):
        sv, dv, e, gse, ga, ss = rings[r]
        off = base + t * CHUNK
        pltpu.async_copy(src_hbm.at[pl.ds(off, CHUNK)], sv, gse)
        pltpu.async_copy(dst_hbm.at[pl.ds(off, CHUNK)], dv, gse)
        pltpu.async_copy(epr_hbm.at[pl.ds(off, CHUNK)], e, gse)

    def drain_head(t, r):
        sv, dv, e, gse, ga, ss = rings[r]
        off = base + t * CHUNK
        pltpu.make_async_copy(src_hbm.at[pl.ds(off, CHUNK)], sv, gse).wait()
        pltpu.make_async_copy(dst_hbm.at[pl.ds(off, CHUNK)], dv, gse).wait()
        pltpu.make_async_copy(epr_hbm.at[pl.ds(off, CHUNK)], e, gse).wait()

    def issue_adds(r):
        sv, dv, e, gse, ga, ss = rings[r]
        pltpu.async_copy(a_hbm.at[sv], e, ga, add=True)
        pltpu.async_copy(b_hbm.at[dv], e, ga, add=True)

    def drain_adds(r):
        sv, dv, e, gse, ga, ss = rings[r]
        pltpu.make_async_copy(a_hbm.at[sv], e, ga).wait()
        pltpu.make_async_copy(b_hbm.at[dv], e, ga).wait()

    def compute(r):
        e = rings[r][2]

        def row(i, carry):
            for j in range(D // LANES):
                sl = pl.ds(j * LANES, LANES)
                e[i, sl] = jnp.maximum(e[i, sl], 0.0)
            return carry

        lax.fori_loop(0, CHUNK, row, 0)

    def scatter(r):
        sv, dv, e, gse, ga, ss = rings[r]
        pltpu.async_copy(e, agg_sh.at[dv], ss, add=True)

    def wait_scatter(r):
        sv, dv, e, gse, ga, ss = rings[r]
        pltpu.make_async_copy(e, agg_sh.at[dv], ss).wait()

    def steady(t, r, nxt_adds=True, nxt2_head=True, wait_sc=True):
        drain_adds(r)
        if nxt_adds:
            drain_head(t + 1, (r + 1) % 3)
            issue_adds((r + 1) % 3)
        if nxt2_head:
            if wait_sc:
                wait_scatter((r + 2) % 3)
            issue_head(t + 2, (r + 2) % 3)
        compute(r)
        scatter(r)

    pltpu.sync_copy(zeros_hbm, agg_sh.at[pl.ds(s * RPS, RPS)])
    plsc.subcore_barrier()

    issue_head(0, 0)
    drain_head(0, 0)
    issue_adds(0)
    issue_head(1, 1)
    steady(0, 0, wait_sc=False)
    steady(1, 1)

    def loop_body(k, carry):
        t0 = 2 + 3 * k
        steady(t0, 2)
        steady(t0 + 1, 0)
        steady(t0 + 2, 1)
        return carry

    lax.fori_loop(0, (NCHUNK - 5) // 3, loop_body, 0)
    steady(NCHUNK - 3, 2)
    steady(NCHUNK - 2, 0, nxt2_head=False)
    steady(NCHUNK - 1, 1, nxt_adds=False, nxt2_head=False)
    wait_scatter(2)
    wait_scatter(0)
    wait_scatter(1)
    plsc.subcore_barrier()

    row0 = c * N_PAD + s * RPS
    pltpu.sync_copy(agg_sh.at[pl.ds(s * RPS, RPS)],
                    out_hbm.at[pl.ds(row0, RPS)])

  return edge_pass


def _edge_pass(a, b, epr, src, dst, zeros):
    return _make_edge_pass()(a, b, epr, src, dst, zeros)


def kernel(x, edge_index, batch, edge_attr, pos,
           Wm1, bm1, Wu1, bu1, Wm2, bm2, Wu2, bu2, Wout, bout):
    del pos
    src = edge_index[0].astype(jnp.int32)
    dst = edge_index[1].astype(jnp.int32)
    batch_i32 = batch.astype(jnp.int32)
    zeros = jnp.zeros((RPS, D), _f32)

    ws1, wd1, we1 = Wm1[:D], Wm1[D:2 * D], Wm1[2 * D:]
    ws2, wd2, we2 = Wm2[:D], Wm2[D:2 * D], Wm2[2 * D:]
    wx1, wa1 = Wu1[:D], Wu1[D:]
    wx2, wa2 = Wu2[:D], Wu2[D:]

    epr1 = _edge_proj(edge_attr, we1)
    a1, b1 = _node_proj(x, ws1, wd1, bm1)
    parts1 = _edge_pass(a1, b1, epr1, src, dst, zeros)

    epr2 = _edge_proj(edge_attr, we2)
    h1, a2, b2 = _update_proj(x, parts1, wx1, wa1, bu1, ws2, wd2, bm2)
    parts2 = _edge_pass(a2, b2, epr2, src, dst, zeros)

    return _update_pool(h1, parts2, wx2, wa2, bu2, batch_i32, Wout, bout)


# speedup vs baseline: 5.3317x; 1.1698x over previous
import functools

import jax
import jax.numpy as jnp
from jax import lax
from jax.experimental import pallas as pl
from jax.experimental.pallas import tpu as pltpu
from jax.experimental.pallas import tpu_sc as plsc

N_NODES = 10000
N_EDGES = 320000
D = 128
D_EDGE = 16
G = 64
LANES = 16

NW = 32
EPW = N_EDGES // NW
CHUNK = 80
NCHUNK = EPW // CHUNK
NSUB = 16
N_PAD = 10240
RPS = N_PAD // NSUB

_f32 = jnp.float32


def _proj_body(x_ref, ws_ref, wd_ref, bm_ref, a_ref, b_ref):
    x = x_ref[...]
    a_ref[...] = jnp.dot(x, ws_ref[...], preferred_element_type=_f32)
    b_ref[...] = jnp.dot(x, wd_ref[...], preferred_element_type=_f32) + bm_ref[...]


def _node_proj(x, ws, wd, bm):
    return pl.pallas_call(
        _proj_body,
        out_shape=[
            jax.ShapeDtypeStruct((N_NODES, D), _f32),
            jax.ShapeDtypeStruct((N_NODES, D), _f32),
        ],
    )(x, ws, wd, bm.reshape(1, D))


_EBLK = 8000


def _epr_body(ea_ref, w_ref, e_ref):
    e_ref[...] = jnp.dot(ea_ref[...], w_ref[...], preferred_element_type=_f32)


def _edge_proj(edge_attr, we):
    return pl.pallas_call(
        _epr_body,
        grid=(N_EDGES // _EBLK,),
        in_specs=[
            pl.BlockSpec((_EBLK, D_EDGE), lambda i: (i, 0)),
            pl.BlockSpec((D_EDGE, D), lambda i: (0, 0)),
        ],
        out_specs=pl.BlockSpec((_EBLK, D), lambda i: (i, 0)),
        out_shape=jax.ShapeDtypeStruct((N_EDGES, D), _f32),
    )(edge_attr, we)


def _update_proj_body(x_ref, p_ref, wx_ref, wa_ref, bu_ref, ws_ref, wd_ref,
                      bm_ref, h_ref, a_ref, b_ref):
    agg = p_ref[0, :N_NODES] + p_ref[1, :N_NODES]
    h = (jnp.dot(x_ref[...], wx_ref[...], preferred_element_type=_f32)
         + jnp.dot(agg, wa_ref[...], preferred_element_type=_f32)
         + bu_ref[...])
    h = jnp.maximum(h, 0.0)
    h_ref[...] = h
    a_ref[...] = jnp.dot(h, ws_ref[...], preferred_element_type=_f32)
    b_ref[...] = jnp.dot(h, wd_ref[...], preferred_element_type=_f32) + bm_ref[...]


def _update_proj(x, parts, wx, wa, bu, ws, wd, bm):
    return pl.pallas_call(
        _update_proj_body,
        out_shape=[
            jax.ShapeDtypeStruct((N_NODES, D), _f32),
            jax.ShapeDtypeStruct((N_NODES, D), _f32),
            jax.ShapeDtypeStruct((N_NODES, D), _f32),
        ],
    )(x, parts.reshape(2, N_PAD, D), wx, wa, bu.reshape(1, D),
      ws, wd, bm.reshape(1, D))


def _update_pool_body(x_ref, p_ref, wx_ref, wa_ref, bu_ref, batch_ref,
                      wout_ref, bout_ref, o_ref):
    agg = p_ref[0, :N_NODES] + p_ref[1, :N_NODES]
    h = (jnp.dot(x_ref[...], wx_ref[...], preferred_element_type=_f32)
         + jnp.dot(agg, wa_ref[...], preferred_element_type=_f32)
         + bu_ref[...])
    h = jnp.maximum(h, 0.0)
    onehot = (batch_ref[...] ==
              lax.broadcasted_iota(jnp.int32, (G, N_NODES), 0)).astype(_f32)
    sums = jnp.dot(onehot, h, preferred_element_type=_f32)
    counts = jnp.sum(onehot, axis=1)
    pooled = sums / jnp.maximum(counts, 1.0)[:, None]
    o_ref[...] = jnp.dot(pooled, wout_ref[...], preferred_element_type=_f32) + bout_ref[...]


def _update_pool(x, parts, wx, wa, bu, batch_i32, wout, bout):
    return pl.pallas_call(
        _update_pool_body,
        out_shape=jax.ShapeDtypeStruct((G, wout.shape[1]), _f32),
    )(x, parts.reshape(2, N_PAD, D), wx, wa, bu.reshape(1, D),
      batch_i32.reshape(1, N_NODES), wout, bout.reshape(1, wout.shape[1]))


@functools.lru_cache(maxsize=1)
def _make_edge_pass():
  mesh = plsc.VectorSubcoreMesh(core_axis_name="c", subcore_axis_name="s")

  ring_types = []
  for _ in range(3):
      ring_types += [pltpu.VMEM((CHUNK,), jnp.int32),
                     pltpu.VMEM((CHUNK,), jnp.int32),
                     pltpu.VMEM((CHUNK, D), _f32)]

  @functools.partial(
      pl.kernel,
      out_type=jax.ShapeDtypeStruct((2 * N_PAD, D), _f32),
      mesh=mesh,
      scratch_types=ring_types + [
          pltpu.VMEM_SHARED((N_PAD, D), _f32),
      ] + [pltpu.SemaphoreType.DMA] * 9,
  )
  def edge_pass(a_hbm, b_hbm, epr_hbm, src_hbm, dst_hbm, zeros_hbm, out_hbm,
                s0, d0, e0, s1, d1, e1, s2, d2, e2,
                agg_sh, gse0, gse1, gse2, ga0, ga1, ga2, ss0, ss1, ss2):
    c = lax.axis_index("c")
    s = lax.axis_index("s")
    wid = s * 2 + c
    base = wid * EPW

    rings = ((s0, d0, e0, gse0, ga0, ss0),
             (s1, d1, e1, gse1, ga1, ss1),
             (s2, d2, e2, gse2, ga2, ss2))

    def issue_head(t, r):
        sv, dv, e, gse, ga, ss = rings[r]
        off = base + t * CHUNK
        pltpu.async_copy(src_hbm.at[pl.ds(off, CHUNK)], sv, gse)
        pltpu.async_copy(dst_hbm.at[pl.ds(off, CHUNK)], dv, gse)
        pltpu.async_copy(epr_hbm.at[pl.ds(off, CHUNK)], e, gse)

    def drain_head(t, r):
        sv, dv, e, gse, ga, ss = rings[r]
        off = base + t * CHUNK
        pltpu.make_async_copy(src_hbm.at[pl.ds(off, CHUNK)], sv, gse).wait()
        pltpu.make_async_copy(dst_hbm.at[pl.ds(off, CHUNK)], dv, gse).wait()
        pltpu.make_async_copy(epr_hbm.at[pl.ds(off, CHUNK)], e, gse).wait()

    def issue_adds(r):
        sv, dv, e, gse, ga, ss = rings[r]
        pltpu.async_copy(a_hbm.at[sv], e, ga, add=True)
        pltpu.async_copy(b_hbm.at[dv], e, ga, add=True)

    def drain_adds(r):
        sv, dv, e, gse, ga, ss = rings[r]
        pltpu.make_async_copy(a_hbm.at[sv], e, ga).wait()
        pltpu.make_async_copy(b_hbm.at[dv], e, ga).wait()

    def compute(r):
        e = rings[r][2]

        def row(i, carry):
            for j in range(D // LANES):
                sl = pl.ds(j * LANES, LANES)
                e[i, sl] = jnp.maximum(e[i, sl], 0.0)
            return carry

        lax.fori_loop(0, CHUNK, row, 0)

    def scatter(r):
        sv, dv, e, gse, ga, ss = rings[r]
        pltpu.async_copy(e, agg_sh.at[dv], ss, add=True)

    def wait_scatter(r):
        sv, dv, e, gse, ga, ss = rings[r]
        pltpu.make_async_copy(e, agg_sh.at[dv], ss).wait()

    def steady(t, r, nxt_adds=True, nxt2_head=True, wait_sc=True):
        drain_adds(r)
        if nxt_adds:
            drain_head(t + 1, (r + 1) % 3)
            issue_adds((r + 1) % 3)
        if nxt2_head:
            if wait_sc:
                wait_scatter((r + 2) % 3)
            issue_head(t + 2, (r + 2) % 3)
        compute(r)
        scatter(r)

    pltpu.sync_copy(zeros_hbm, agg_sh.at[pl.ds(s * RPS, RPS)])
    plsc.subcore_barrier()

    issue_head(0, 0)
    drain_head(0, 0)
    issue_adds(0)
    issue_head(1, 1)
    steady(0, 0, wait_sc=False)
    steady(1, 1)

    def loop_body(k, carry):
        t0 = 2 + 3 * k
        steady(t0, 2)
        steady(t0 + 1, 0)
        steady(t0 + 2, 1)
        return carry

    lax.fori_loop(0, (NCHUNK - 5) // 3, loop_body, 0)
    steady(NCHUNK - 3, 2)
    steady(NCHUNK - 2, 0, nxt2_head=False)
    steady(NCHUNK - 1, 1, nxt_adds=False, nxt2_head=False)
    wait_scatter(2)
    wait_scatter(0)
    wait_scatter(1)
    plsc.subcore_barrier()

    row0 = c * N_PAD + s * RPS
    pltpu.sync_copy(agg_sh.at[pl.ds(s * RPS, RPS)],
                    out_hbm.at[pl.ds(row0, RPS)])

  return edge_pass


def _edge_pass(a, b, epr, src, dst, zeros):
    return _make_edge_pass()(a, b, epr, src, dst, zeros)


def kernel(x, edge_index, batch, edge_attr, pos,
           Wm1, bm1, Wu1, bu1, Wm2, bm2, Wu2, bu2, Wout, bout):
    del pos
    src = edge_index[0].astype(jnp.int32)
    dst = edge_index[1].astype(jnp.int32)
    batch_i32 = batch.astype(jnp.int32)
    zeros = jnp.zeros((RPS, D), _f32)

    ws1, wd1, we1 = Wm1[:D], Wm1[D:2 * D], Wm1[2 * D:]
    ws2, wd2, we2 = Wm2[:D], Wm2[D:2 * D], Wm2[2 * D:]
    wx1, wa1 = Wu1[:D], Wu1[D:]
    wx2, wa2 = Wu2[:D], Wu2[D:]

    epr1 = _edge_proj(edge_attr, we1)
    a1, b1 = _node_proj(x, ws1, wd1, bm1)
    parts1 = _edge_pass(a1, b1, epr1, src, dst, zeros)

    epr2 = _edge_proj(edge_attr, we2)
    h1, a2, b2 = _update_proj(x, parts1, wx1, wa1, bu1, ws2, wd2, bm2)
    parts2 = _edge_pass(a2, b2, epr2, src, dst, zeros)

    return _update_pool(h1, parts2, wx2, wa2, bu2, batch_i32, Wout, bout)


# speedup vs baseline: 6.4826x; 1.2158x over previous
import functools

import jax
import jax.numpy as jnp
from jax import lax
from jax.experimental import pallas as pl
from jax.experimental.pallas import tpu as pltpu
from jax.experimental.pallas import tpu_sc as plsc

N_NODES = 10000
N_EDGES = 320000
D = 128
D_EDGE = 16
G = 64
LANES = 16

NW = 32
EPW = N_EDGES // NW
CHUNK = 80
NCHUNK = EPW // CHUNK
NSUB = 16
N_PAD = 10240
RPS = N_PAD // NSUB

_f32 = jnp.float32


def _proj_body(x_ref, ws_ref, wd_ref, bm_ref, a_ref, b_ref):
    x = x_ref[...]
    a_ref[...] = jnp.dot(x, ws_ref[...], preferred_element_type=_f32)
    b_ref[...] = jnp.dot(x, wd_ref[...], preferred_element_type=_f32) + bm_ref[...]


def _node_proj(x, ws, wd, bm):
    return pl.pallas_call(
        _proj_body,
        out_shape=[
            jax.ShapeDtypeStruct((N_NODES, D), _f32),
            jax.ShapeDtypeStruct((N_NODES, D), _f32),
        ],
    )(x, ws, wd, bm.reshape(1, D))


_EBLK = 12800


def _epr_body(eat_ref, w_ref, e_ref):
    e_ref[...] = lax.dot_general(eat_ref[...], w_ref[...],
                                 (((0,), (0,)), ((), ())),
                                 preferred_element_type=_f32)


def _edge_proj(edge_attr, we):
    eat = edge_attr.T
    return pl.pallas_call(
        _epr_body,
        grid=(N_EDGES // _EBLK,),
        in_specs=[
            pl.BlockSpec((D_EDGE, _EBLK), lambda i: (0, i)),
            pl.BlockSpec((D_EDGE, D), lambda i: (0, 0)),
        ],
        out_specs=pl.BlockSpec((_EBLK, D), lambda i: (i, 0)),
        out_shape=jax.ShapeDtypeStruct((N_EDGES, D), _f32),
    )(eat, we)


def _update_proj_body(x_ref, p_ref, wx_ref, wa_ref, bu_ref, ws_ref, wd_ref,
                      bm_ref, h_ref, a_ref, b_ref):
    agg = p_ref[0, :N_NODES] + p_ref[1, :N_NODES]
    h = (jnp.dot(x_ref[...], wx_ref[...], preferred_element_type=_f32)
         + jnp.dot(agg, wa_ref[...], preferred_element_type=_f32)
         + bu_ref[...])
    h = jnp.maximum(h, 0.0)
    h_ref[...] = h
    a_ref[...] = jnp.dot(h, ws_ref[...], preferred_element_type=_f32)
    b_ref[...] = jnp.dot(h, wd_ref[...], preferred_element_type=_f32) + bm_ref[...]


def _update_proj(x, parts, wx, wa, bu, ws, wd, bm):
    return pl.pallas_call(
        _update_proj_body,
        out_shape=[
            jax.ShapeDtypeStruct((N_NODES, D), _f32),
            jax.ShapeDtypeStruct((N_NODES, D), _f32),
            jax.ShapeDtypeStruct((N_NODES, D), _f32),
        ],
    )(x, parts.reshape(2, N_PAD, D), wx, wa, bu.reshape(1, D),
      ws, wd, bm.reshape(1, D))


def _update_pool_body(x_ref, p_ref, wx_ref, wa_ref, bu_ref, batch_ref,
                      wout_ref, bout_ref, o_ref):
    agg = p_ref[0, :N_NODES] + p_ref[1, :N_NODES]
    h = (jnp.dot(x_ref[...], wx_ref[...], preferred_element_type=_f32)
         + jnp.dot(agg, wa_ref[...], preferred_element_type=_f32)
         + bu_ref[...])
    h = jnp.maximum(h, 0.0)
    onehot = (batch_ref[...] ==
              lax.broadcasted_iota(jnp.int32, (G, N_NODES), 0)).astype(_f32)
    sums = jnp.dot(onehot, h, preferred_element_type=_f32)
    counts = jnp.sum(onehot, axis=1)
    pooled = sums / jnp.maximum(counts, 1.0)[:, None]
    o_ref[...] = jnp.dot(pooled, wout_ref[...], preferred_element_type=_f32) + bout_ref[...]


def _update_pool(x, parts, wx, wa, bu, batch_i32, wout, bout):
    return pl.pallas_call(
        _update_pool_body,
        out_shape=jax.ShapeDtypeStruct((G, wout.shape[1]), _f32),
    )(x, parts.reshape(2, N_PAD, D), wx, wa, bu.reshape(1, D),
      batch_i32.reshape(1, N_NODES), wout, bout.reshape(1, wout.shape[1]))


@functools.lru_cache(maxsize=1)
def _make_edge_pass():
  mesh = plsc.VectorSubcoreMesh(core_axis_name="c", subcore_axis_name="s")

  ring_types = []
  for _ in range(3):
      ring_types += [pltpu.VMEM((CHUNK,), jnp.int32),
                     pltpu.VMEM((CHUNK,), jnp.int32),
                     pltpu.VMEM((CHUNK, D), _f32)]

  @functools.partial(
      pl.kernel,
      out_type=jax.ShapeDtypeStruct((2 * N_PAD, D), _f32),
      mesh=mesh,
      scratch_types=ring_types + [
          pltpu.VMEM_SHARED((N_PAD, D), _f32),
      ] + [pltpu.SemaphoreType.DMA] * 9,
  )
  def edge_pass(a_hbm, b_hbm, epr_hbm, src_hbm, dst_hbm, zeros_hbm, out_hbm,
                s0, d0, e0, s1, d1, e1, s2, d2, e2,
                agg_sh, gse0, gse1, gse2, ga0, ga1, ga2, ss0, ss1, ss2):
    c = lax.axis_index("c")
    s = lax.axis_index("s")
    wid = s * 2 + c
    base = wid * EPW

    rings = ((s0, d0, e0, gse0, ga0, ss0),
             (s1, d1, e1, gse1, ga1, ss1),
             (s2, d2, e2, gse2, ga2, ss2))

    def issue_head(t, r):
        sv, dv, e, gse, ga, ss = rings[r]
        off = base + t * CHUNK
        pltpu.async_copy(src_hbm.at[pl.ds(off, CHUNK)], sv, gse)
        pltpu.async_copy(dst_hbm.at[pl.ds(off, CHUNK)], dv, gse)
        pltpu.async_copy(epr_hbm.at[pl.ds(off, CHUNK)], e, gse)

    def drain_head(t, r):
        sv, dv, e, gse, ga, ss = rings[r]
        off = base + t * CHUNK
        pltpu.make_async_copy(src_hbm.at[pl.ds(off, CHUNK)], sv, gse).wait()
        pltpu.make_async_copy(dst_hbm.at[pl.ds(off, CHUNK)], dv, gse).wait()
        pltpu.make_async_copy(epr_hbm.at[pl.ds(off, CHUNK)], e, gse).wait()

    def issue_adds(r):
        sv, dv, e, gse, ga, ss = rings[r]
        pltpu.async_copy(a_hbm.at[sv], e, ga, add=True)
        pltpu.async_copy(b_hbm.at[dv], e, ga, add=True)

    def drain_adds(r):
        sv, dv, e, gse, ga, ss = rings[r]
        pltpu.make_async_copy(a_hbm.at[sv], e, ga).wait()
        pltpu.make_async_copy(b_hbm.at[dv], e, ga).wait()

    def compute(r):
        e = rings[r][2]

        def row(i, carry):
            for j in range(D // LANES):
                sl = pl.ds(j * LANES, LANES)
                e[i, sl] = jnp.maximum(e[i, sl], 0.0)
            return carry

        lax.fori_loop(0, CHUNK, row, 0)

    def scatter(r):
        sv, dv, e, gse, ga, ss = rings[r]
        pltpu.async_copy(e, agg_sh.at[dv], ss, add=True)

    def wait_scatter(r):
        sv, dv, e, gse, ga, ss = rings[r]
        pltpu.make_async_copy(e, agg_sh.at[dv], ss).wait()

    def steady(t, r, nxt_adds=True, nxt2_head=True, wait_sc=True):
        drain_adds(r)
        if nxt_adds:
            drain_head(t + 1, (r + 1) % 3)
            issue_adds((r + 1) % 3)
        if nxt2_head:
            if wait_sc:
                wait_scatter((r + 2) % 3)
            issue_head(t + 2, (r + 2) % 3)
        compute(r)
        scatter(r)

    pltpu.sync_copy(zeros_hbm, agg_sh.at[pl.ds(s * RPS, RPS)])
    plsc.subcore_barrier()

    issue_head(0, 0)
    drain_head(0, 0)
    issue_adds(0)
    issue_head(1, 1)
    steady(0, 0, wait_sc=False)
    steady(1, 1)

    def loop_body(k, carry):
        t0 = 2 + 3 * k
        steady(t0, 2)
        steady(t0 + 1, 0)
        steady(t0 + 2, 1)
        return carry

    lax.fori_loop(0, (NCHUNK - 5) // 3, loop_body, 0)
    steady(NCHUNK - 3, 2)
    steady(NCHUNK - 2, 0, nxt2_head=False)
    steady(NCHUNK - 1, 1, nxt_adds=False, nxt2_head=False)
    wait_scatter(2)
    wait_scatter(0)
    wait_scatter(1)
    plsc.subcore_barrier()

    row0 = c * N_PAD + s * RPS
    pltpu.sync_copy(agg_sh.at[pl.ds(s * RPS, RPS)],
                    out_hbm.at[pl.ds(row0, RPS)])

  return edge_pass


def _edge_pass(a, b, epr, src, dst, zeros):
    return _make_edge_pass()(a, b, epr, src, dst, zeros)


def kernel(x, edge_index, batch, edge_attr, pos,
           Wm1, bm1, Wu1, bu1, Wm2, bm2, Wu2, bu2, Wout, bout):
    del pos
    src = edge_index[0].astype(jnp.int32)
    dst = edge_index[1].astype(jnp.int32)
    batch_i32 = batch.astype(jnp.int32)
    zeros = jnp.zeros((RPS, D), _f32)

    ws1, wd1, we1 = Wm1[:D], Wm1[D:2 * D], Wm1[2 * D:]
    ws2, wd2, we2 = Wm2[:D], Wm2[D:2 * D], Wm2[2 * D:]
    wx1, wa1 = Wu1[:D], Wu1[D:]
    wx2, wa2 = Wu2[:D], Wu2[D:]

    epr1 = _edge_proj(edge_attr, we1)
    a1, b1 = _node_proj(x, ws1, wd1, bm1)
    parts1 = _edge_pass(a1, b1, epr1, src, dst, zeros)

    epr2 = _edge_proj(edge_attr, we2)
    h1, a2, b2 = _update_proj(x, parts1, wx1, wa1, bu1, ws2, wd2, bm2)
    parts2 = _edge_pass(a2, b2, epr2, src, dst, zeros)

    return _update_pool(h1, parts2, wx2, wa2, bu2, batch_i32, Wout, bout)
